# Initial kernel scaffold; baseline (speedup 1.0000x reference)
import jax
import jax.numpy as jnp
from jax.experimental import pallas as pl


def kernel(input, table):
    raise NotImplementedError("write your pallas kernel here")


# speedup vs baseline: 3.3015x; 3.3015x over previous
import functools

import jax
import jax.numpy as jnp
from jax import lax
from jax.experimental import pallas as pl
from jax.experimental.pallas import tpu as pltpu
from jax.experimental.pallas import tpu_sc as plsc

PAD_INDEX = 0
BATCH, SEQ = 4096, 200
NUM_EMB, DIM = 256, 128
NC, NS, L = 2, 16, 16
NW = NC * NS
RPW = BATCH // NW


def _mask(x):
    return lax.shift_right_logical(x | (0 - x), 31).astype(jnp.int32)


def _pos_chunks(inp_v, r, pos_v):
    carry = jnp.int32(0)
    for j in range(11):
        x = inp_v[r, pl.ds(j * L, L)]
        m = _mask(x)
        c = plsc.cumsum(m) + carry
        pos_v[j // 8, pl.ds((j % 8) * L, L)] = c * m
        carry = carry + jnp.sum(m)
    x = inp_v[r, pl.ds(176, L)]
    m = _mask(x)
    c = plsc.cumsum(m) + carry
    pos_v[1, pl.ds(48, L)] = c * m
    lane = lax.iota(jnp.int32, L)
    first8 = lax.shift_right_logical(lane - 8, 31).astype(jnp.int32)
    carry_t = carry + jnp.sum(m * first8)
    x = inp_v[r, pl.ds(184, L)]
    m = _mask(x)
    c = plsc.cumsum(m) + carry_t
    pos_v[1, pl.ds(56, L)] = c * m


def kernel(input, table):
    mesh = plsc.VectorSubcoreMesh(core_axis_name="c", subcore_axis_name="s")

    @functools.partial(
        pl.kernel,
        out_type=jax.ShapeDtypeStruct((BATCH, SEQ, DIM), jnp.float32),
        mesh=mesh,
        compiler_params=pltpu.CompilerParams(needs_layout_passes=False),
        scratch_types=[
            pltpu.VMEM((RPW, SEQ), jnp.int32),
            pltpu.VMEM((2, 128), jnp.int32),
            pltpu.VMEM((SEQ, DIM), jnp.float32),
            pltpu.SemaphoreType.DMA,
        ],
    )
    def run(inp_hbm, tbl_hbm, out_hbm, inp_v, pos_v, rows_v, sem):
        wid = lax.axis_index("s") * NC + lax.axis_index("c")
        base = wid * RPW
        pltpu.sync_copy(inp_hbm.at[pl.ds(base, RPW)], inp_v)

        def row_body(r, _):
            _pos_chunks(inp_v, r, pos_v)
            pltpu.async_copy(
                tbl_hbm.at[pos_v.at[0]], rows_v.at[pl.ds(0, 128)], sem
            ).wait()
            pltpu.async_copy(
                tbl_hbm.at[pos_v.at[1, pl.ds(0, 72)]],
                rows_v.at[pl.ds(128, 72)],
                sem,
            ).wait()
            pltpu.sync_copy(rows_v, out_hbm.at[base + r])
            return 0

        lax.fori_loop(0, RPW, row_body, 0)

    return run(input, table)
